# baseline (speedup 1.0000x reference)
import functools

import jax
import jax.numpy as jnp
from jax import lax
from jax.experimental import pallas as pl
from jax.experimental.pallas import tpu as pltpu
from jax.experimental.pallas import tpu_sc as plsc

_NC = 2
_NS = 16
_NW = _NC * _NS
_K = 128
_NBUF = 4

_V = 1000000
_D = 64
_RS = 512
_FULL = _V // _RS
_PER_W = _FULL // _NW
_TAIL_R0 = _FULL * _RS
_TAIL_N = _V - _TAIL_R0
_BLK = _RS * _D // 2048


def _tr_blocks(ibuf, obuf, buf, nrows):
    cis = [lax.iota(jnp.int32, 16) + c0 for c0 in range(0, _D, 16)]

    @pl.loop(0, nrows, unroll=8)
    def _row(q):
        b = q // 32
        s2 = (q % 32) // 2
        l0 = (q % 2) * 64
        qv = jnp.full((16,), q, jnp.int32)
        for i, c0 in enumerate(range(0, _D, 16)):
            v = plsc.load_gather(ibuf, [cis[i], qv])
            obuf[buf, b, s2, pl.ds(l0 + c0, 16)] = v


def _detr_body(wt_hbm, wtail_hbm, w3_hbm, ibuf, tbuf, obuf, sem0, sem1):
    wid = lax.axis_index("s") * _NC + lax.axis_index("c")
    base = wid * _PER_W

    def stage_in(slab):
        pltpu.sync_copy(wt_hbm.at[:, pl.ds(slab * _RS, _RS)], ibuf)

    def stage_out(slab, buf, sem):
        pltpu.async_copy(
            obuf.at[buf], w3_hbm.at[pl.ds(slab * _BLK, _BLK)], sem
        )

    def wait_out(buf, sem):
        pltpu.make_async_copy(
            obuf.at[buf], w3_hbm.at[pl.ds(0, _BLK)], sem
        ).wait()

    for b in range(2):
        stage_in(base + b)
        _tr_blocks(ibuf, obuf, b, _RS)
        stage_out(base + b, b, (sem0, sem1)[b])

    @pl.loop(2, _PER_W - 1, step=2)
    def _pair(k):
        for b in range(2):
            sem = (sem0, sem1)[b]
            wait_out(b, sem)
            stage_in(base + k + b)
            _tr_blocks(ibuf, obuf, b, _RS)
            stage_out(base + k + b, b, sem)

    wait_out(0, sem0)
    stage_in(base + _PER_W - 1)
    _tr_blocks(ibuf, obuf, 0, _RS)
    stage_out(base + _PER_W - 1, 0, sem0)

    @pl.when(wid == 0)
    def _extra():
        wait_out(1, sem1)
        stage_in(_NW * _PER_W)
        _tr_blocks(ibuf, obuf, 1, _RS)
        stage_out(_NW * _PER_W, 1, sem1)

    @pl.when(wid == 1)
    def _tail():
        wait_out(1, sem1)
        pltpu.sync_copy(wtail_hbm, tbuf)
        _tr_blocks(tbuf, obuf, 1, _TAIL_N)
        nb = _TAIL_N * _D // 2048
        pltpu.async_copy(
            obuf.at[1, pl.ds(0, nb)],
            w3_hbm.at[pl.ds(_TAIL_R0 * _D // 2048, nb)],
            sem1,
        ).wait()

    wait_out(0, sem0)
    pl.when(wid != 1)(lambda: wait_out(1, sem1))


def _emb_body(idx_hbm, table_hbm, out_hbm, idx_v, rows_v, *gsems):
    wid = lax.axis_index("s") * _NC + lax.axis_index("c")
    chunks = idx_hbm.shape[1]
    pltpu.sync_copy(idx_hbm.at[wid], idx_v)
    row0 = wid * chunks * _K

    @pl.loop(0, chunks, step=_NBUF)
    def _group(g0):
        cps = []
        for b in range(_NBUF):
            g = g0 + b
            cps.append(
                pltpu.async_copy(table_hbm.at[idx_v.at[g]], rows_v.at[b], gsems[b])
            )
        for b in range(_NBUF):
            cps[b].wait()
            pltpu.sync_copy(
                rows_v.at[b], out_hbm.at[pl.ds(row0 + (g0 + b) * _K, _K)]
            )


def kernel(x, weight):
    b, s = x.shape
    v, d = weight.shape
    assert (v, d) == (_V, _D)
    n = b * s
    assert n % (_NW * _K) == 0
    chunks = n // (_NW * _K)

    mesh = plsc.VectorSubcoreMesh(core_axis_name="c", subcore_axis_name="s")

    detr = pl.kernel(
        _detr_body,
        out_type=jax.ShapeDtypeStruct((_V * _D // 2048, 16, 128), jnp.float32),
        mesh=mesh,
        scratch_types=[
            pltpu.VMEM((_D, _RS), jnp.float32),
            pltpu.VMEM((_D, _TAIL_N), jnp.float32),
            pltpu.VMEM((2, _BLK, 16, 128), jnp.float32),
            pltpu.SemaphoreType.DMA,
            pltpu.SemaphoreType.DMA,
        ],
        compiler_params=pltpu.CompilerParams(
            use_tc_tiling_on_sc=True, needs_layout_passes=False
        ),
    )
    wtail_t = lax.slice(weight, (_TAIL_R0, 0), (_V, _D)).T
    w3 = detr(weight.T, wtail_t)
    wrm = w3.reshape(_V, _D)

    flat = x.reshape(-1).astype(jnp.int32)
    idx3 = flat.reshape(_NW, chunks, _K)

    run = pl.kernel(
        _emb_body,
        out_type=jax.ShapeDtypeStruct((n, d), jnp.float32),
        mesh=mesh,
        scratch_types=[
            pltpu.VMEM((chunks, _K), jnp.int32),
            pltpu.VMEM((_NBUF, _K, d), jnp.float32),
        ]
        + [pltpu.SemaphoreType.DMA] * _NBUF,
        compiler_params=pltpu.CompilerParams(use_tc_tiling_on_sc=False),
    )
    out = run(idx3, wrm)
    return out.reshape(b, s, d)


# speedup vs baseline: 1.9341x; 1.9341x over previous
import functools

import jax
import jax.numpy as jnp
from jax import lax
from jax.experimental import pallas as pl
from jax.experimental.pallas import tpu as pltpu
from jax.experimental.pallas import tpu_sc as plsc

_NC = 2
_NS = 16
_NW = _NC * _NS
_K = 128
_NBUF = 4

_V = 1000000
_D = 64
_RS = 512
_FULL = _V // _RS
_PER_W = _FULL // _NW
_TAIL_R0 = _FULL * _RS
_TAIL_N = _V - _TAIL_R0
_SLABW = _RS * _D


def _tr_blocks(ibuf, obuf, buf, nrows):
    iot = lax.iota(jnp.int32, 16)
    rots = [(iot + d) % 16 for d in range(16)]
    rv64 = [rots[d] * 64 + iot for d in range(16)]
    bufv = jnp.full((16,), buf, jnp.int32)

    @pl.loop(0, nrows, step=16)
    def _blk(q0):
        qv = jnp.full((16,), q0, jnp.int32)

        @pl.loop(0, _D, step=16)
        def _cblk(c0):
            civ = iot + c0
            base = q0 * 64 + c0
            for d in range(16):
                v = plsc.load_gather(ibuf, [civ, qv + rots[d]])
                plsc.store_scatter(obuf, [bufv, rv64[d] + base], v)


def _detr_body(wt_hbm, wtail_hbm, w3_hbm, ibuf, tbuf, obuf, sem0, sem1):
    wid = lax.axis_index("s") * _NC + lax.axis_index("c")
    base = wid * _PER_W

    def stage_in(slab):
        pltpu.sync_copy(wt_hbm.at[:, pl.ds(slab * _RS, _RS)], ibuf)

    def stage_out(slab, buf, sem):
        pltpu.async_copy(
            obuf.at[buf], w3_hbm.at[pl.ds(slab * _SLABW, _SLABW)], sem
        )

    def wait_out(buf, sem):
        pltpu.make_async_copy(
            obuf.at[buf], w3_hbm.at[pl.ds(0, _SLABW)], sem
        ).wait()

    for b in range(2):
        stage_in(base + b)
        _tr_blocks(ibuf, obuf, b, _RS)
        stage_out(base + b, b, (sem0, sem1)[b])

    @pl.loop(2, _PER_W - 1, step=2)
    def _pair(k):
        for b in range(2):
            sem = (sem0, sem1)[b]
            wait_out(b, sem)
            stage_in(base + k + b)
            _tr_blocks(ibuf, obuf, b, _RS)
            stage_out(base + k + b, b, sem)

    wait_out(0, sem0)
    stage_in(base + _PER_W - 1)
    _tr_blocks(ibuf, obuf, 0, _RS)
    stage_out(base + _PER_W - 1, 0, sem0)

    @pl.when(wid == 0)
    def _extra():
        wait_out(1, sem1)
        stage_in(_NW * _PER_W)
        _tr_blocks(ibuf, obuf, 1, _RS)
        stage_out(_NW * _PER_W, 1, sem1)

    @pl.when(wid == 1)
    def _tail():
        wait_out(1, sem1)
        pltpu.sync_copy(wtail_hbm, tbuf)
        _tr_blocks(tbuf, obuf, 1, _TAIL_N)
        nw = _TAIL_N * _D
        pltpu.async_copy(
            obuf.at[1, pl.ds(0, nw)],
            w3_hbm.at[pl.ds(_TAIL_R0 * _D, nw)],
            sem1,
        ).wait()

    wait_out(0, sem0)
    pl.when(wid != 1)(lambda: wait_out(1, sem1))


def _emb_body(idx_hbm, table_hbm, out_hbm, idx_v, rows_v, *gsems):
    wid = lax.axis_index("s") * _NC + lax.axis_index("c")
    chunks = idx_hbm.shape[1]
    pltpu.sync_copy(idx_hbm.at[wid], idx_v)
    row0 = wid * chunks * _K

    @pl.loop(0, chunks, step=_NBUF)
    def _group(g0):
        cps = []
        for b in range(_NBUF):
            g = g0 + b
            cps.append(
                pltpu.async_copy(table_hbm.at[idx_v.at[g]], rows_v.at[b], gsems[b])
            )
        for b in range(_NBUF):
            cps[b].wait()
            pltpu.sync_copy(
                rows_v.at[b], out_hbm.at[pl.ds(row0 + (g0 + b) * _K, _K)]
            )


def kernel(x, weight):
    b, s = x.shape
    v, d = weight.shape
    assert (v, d) == (_V, _D)
    n = b * s
    assert n % (_NW * _K) == 0
    chunks = n // (_NW * _K)

    mesh = plsc.VectorSubcoreMesh(core_axis_name="c", subcore_axis_name="s")

    detr = pl.kernel(
        _detr_body,
        out_type=jax.ShapeDtypeStruct((_V * _D,), jnp.float32),
        mesh=mesh,
        scratch_types=[
            pltpu.VMEM((_D, _RS), jnp.float32),
            pltpu.VMEM((_D, _TAIL_N), jnp.float32),
            pltpu.VMEM((2, _SLABW), jnp.float32),
            pltpu.SemaphoreType.DMA,
            pltpu.SemaphoreType.DMA,
        ],
        compiler_params=pltpu.CompilerParams(
            use_tc_tiling_on_sc=True, needs_layout_passes=False
        ),
    )
    wtail_t = lax.slice(weight, (_TAIL_R0, 0), (_V, _D)).T
    w3 = detr(weight.T, wtail_t)
    wrm = w3.reshape(_V, _D)

    flat = x.reshape(-1).astype(jnp.int32)
    idx3 = flat.reshape(_NW, chunks, _K)

    run = pl.kernel(
        _emb_body,
        out_type=jax.ShapeDtypeStruct((n, d), jnp.float32),
        mesh=mesh,
        scratch_types=[
            pltpu.VMEM((chunks, _K), jnp.int32),
            pltpu.VMEM((_NBUF, _K, d), jnp.float32),
        ]
        + [pltpu.SemaphoreType.DMA] * _NBUF,
        compiler_params=pltpu.CompilerParams(use_tc_tiling_on_sc=False),
    )
    out = run(idx3, wrm)
    return out.reshape(b, s, d)


# speedup vs baseline: 2.4669x; 1.2754x over previous
import functools

import jax
import jax.numpy as jnp
from jax import lax
from jax.experimental import pallas as pl
from jax.experimental.pallas import tpu as pltpu
from jax.experimental.pallas import tpu_sc as plsc

_NC = 2
_NS = 16
_NW = _NC * _NS
_K = 128
_NBUF = 4

_V = 1000000
_D = 64
_RS = 512
_FULL = _V // _RS
_PER_W = _FULL // _NW
_TAIL_R0 = _FULL * _RS
_TAIL_N = _V - _TAIL_R0
_SLABW = _RS * _D


def _tr_blocks(ibuf, obuf, buf, nrows):
    iot = lax.iota(jnp.int32, 16)
    rots = [(iot + d) % 16 for d in range(16)]
    rv64 = [rots[d] * 64 + iot for d in range(16)]
    bufv = jnp.full((16,), buf, jnp.int32)

    @pl.loop(0, nrows, step=16)
    def _blk(q0):
        qv = jnp.full((16,), q0, jnp.int32)

        @pl.loop(0, _D, step=16)
        def _cblk(c0):
            civ = iot + c0
            base = q0 * 64 + c0
            vs = [plsc.load_gather(ibuf, [civ, qv + rots[d]]) for d in range(16)]
            for d in range(16):
                plsc.store_scatter(obuf, [bufv, rv64[d] + base], vs[d])


def _detr_body(wt_hbm, wtail_hbm, w3_hbm, ibuf, tbuf, obuf, sem0, sem1):
    wid = lax.axis_index("s") * _NC + lax.axis_index("c")
    base = wid * _PER_W

    def stage_in(slab):
        pltpu.sync_copy(wt_hbm.at[:, pl.ds(slab * _RS, _RS)], ibuf)

    def stage_out(slab, buf, sem):
        pltpu.async_copy(
            obuf.at[buf], w3_hbm.at[pl.ds(slab * _SLABW, _SLABW)], sem
        )

    def wait_out(buf, sem):
        pltpu.make_async_copy(
            obuf.at[buf], w3_hbm.at[pl.ds(0, _SLABW)], sem
        ).wait()

    for b in range(2):
        stage_in(base + b)
        _tr_blocks(ibuf, obuf, b, _RS)
        stage_out(base + b, b, (sem0, sem1)[b])

    @pl.loop(2, _PER_W - 1, step=2)
    def _pair(k):
        for b in range(2):
            sem = (sem0, sem1)[b]
            wait_out(b, sem)
            stage_in(base + k + b)
            _tr_blocks(ibuf, obuf, b, _RS)
            stage_out(base + k + b, b, sem)

    wait_out(0, sem0)
    stage_in(base + _PER_W - 1)
    _tr_blocks(ibuf, obuf, 0, _RS)
    stage_out(base + _PER_W - 1, 0, sem0)

    @pl.when(wid == 0)
    def _extra():
        wait_out(1, sem1)
        stage_in(_NW * _PER_W)
        _tr_blocks(ibuf, obuf, 1, _RS)
        stage_out(_NW * _PER_W, 1, sem1)

    @pl.when(wid == 1)
    def _tail():
        wait_out(1, sem1)
        pltpu.sync_copy(wtail_hbm, tbuf)
        _tr_blocks(tbuf, obuf, 1, _TAIL_N)
        nw = _TAIL_N * _D
        pltpu.async_copy(
            obuf.at[1, pl.ds(0, nw)],
            w3_hbm.at[pl.ds(_TAIL_R0 * _D, nw)],
            sem1,
        ).wait()

    wait_out(0, sem0)
    pl.when(wid != 1)(lambda: wait_out(1, sem1))


def _emb_body(idx_hbm, table_hbm, out_hbm, idx_v, rows_v, *gsems):
    wid = lax.axis_index("s") * _NC + lax.axis_index("c")
    chunks = idx_hbm.shape[1]
    pltpu.sync_copy(idx_hbm.at[wid], idx_v)
    row0 = wid * chunks * _K

    @pl.loop(0, chunks, step=_NBUF)
    def _group(g0):
        cps = []
        for b in range(_NBUF):
            g = g0 + b
            cps.append(
                pltpu.async_copy(table_hbm.at[idx_v.at[g]], rows_v.at[b], gsems[b])
            )
        for b in range(_NBUF):
            cps[b].wait()
            pltpu.sync_copy(
                rows_v.at[b], out_hbm.at[pl.ds(row0 + (g0 + b) * _K, _K)]
            )


def kernel(x, weight):
    b, s = x.shape
    v, d = weight.shape
    assert (v, d) == (_V, _D)
    n = b * s
    assert n % (_NW * _K) == 0
    chunks = n // (_NW * _K)

    mesh = plsc.VectorSubcoreMesh(core_axis_name="c", subcore_axis_name="s")

    detr = pl.kernel(
        _detr_body,
        out_type=jax.ShapeDtypeStruct((_V * _D,), jnp.float32),
        mesh=mesh,
        scratch_types=[
            pltpu.VMEM((_D, _RS), jnp.float32),
            pltpu.VMEM((_D, _TAIL_N), jnp.float32),
            pltpu.VMEM((2, _SLABW), jnp.float32),
            pltpu.SemaphoreType.DMA,
            pltpu.SemaphoreType.DMA,
        ],
        compiler_params=pltpu.CompilerParams(
            use_tc_tiling_on_sc=True, needs_layout_passes=False
        ),
    )
    wtail_t = lax.slice(weight, (_TAIL_R0, 0), (_V, _D)).T
    w3 = detr(weight.T, wtail_t)
    wrm = w3.reshape(_V, _D)

    flat = x.reshape(-1).astype(jnp.int32)
    idx3 = flat.reshape(_NW, chunks, _K)

    run = pl.kernel(
        _emb_body,
        out_type=jax.ShapeDtypeStruct((n, d), jnp.float32),
        mesh=mesh,
        scratch_types=[
            pltpu.VMEM((chunks, _K), jnp.int32),
            pltpu.VMEM((_NBUF, _K, d), jnp.float32),
        ]
        + [pltpu.SemaphoreType.DMA] * _NBUF,
        compiler_params=pltpu.CompilerParams(use_tc_tiling_on_sc=False),
    )
    out = run(idx3, wrm)
    return out.reshape(b, s, d)


# speedup vs baseline: 2.5681x; 1.0410x over previous
import functools

import jax
import jax.numpy as jnp
from jax import lax
from jax.experimental import pallas as pl
from jax.experimental.pallas import tpu as pltpu
from jax.experimental.pallas import tpu_sc as plsc

_NC = 2
_NS = 16
_NW = _NC * _NS
_K = 128
_NBUF = 4

_V = 1000000
_D = 64
_RS = 512
_FULL = _V // _RS
_PER_W = _FULL // _NW
_TAIL_R0 = _FULL * _RS
_TAIL_N = _V - _TAIL_R0
_SLABW = _RS * _D


def _tr_blocks(ibuf, obuf, buf, nrows):
    iot = lax.iota(jnp.int32, 16)
    rots = [(iot + d) % 16 for d in range(16)]
    rv64 = [rots[d] * 64 + iot for d in range(16)]
    bufv = jnp.full((16,), buf, jnp.int32)

    @pl.loop(0, nrows, step=16)
    def _blk(q0):
        qv = jnp.full((16,), q0, jnp.int32)

        @pl.loop(0, _D, step=16)
        def _cblk(c0):
            civ = iot + c0
            base = q0 * 64 + c0
            vs = {}
            for d in range(16):
                vs[d] = plsc.load_gather(ibuf, [civ, qv + rots[d]])
                if d >= 2:
                    plsc.store_scatter(obuf, [bufv, rv64[d - 2] + base], vs.pop(d - 2))
            for d in (14, 15):
                plsc.store_scatter(obuf, [bufv, rv64[d] + base], vs.pop(d))


def _detr_body(wt_hbm, wtail_hbm, w3_hbm, ibuf, tbuf, obuf, sem0, sem1):
    wid = lax.axis_index("s") * _NC + lax.axis_index("c")
    base = wid * _PER_W

    def stage_in(slab):
        pltpu.sync_copy(wt_hbm.at[:, pl.ds(slab * _RS, _RS)], ibuf)

    def stage_out(slab, buf, sem):
        pltpu.async_copy(
            obuf.at[buf], w3_hbm.at[pl.ds(slab * _SLABW, _SLABW)], sem
        )

    def wait_out(buf, sem):
        pltpu.make_async_copy(
            obuf.at[buf], w3_hbm.at[pl.ds(0, _SLABW)], sem
        ).wait()

    for b in range(2):
        stage_in(base + b)
        _tr_blocks(ibuf, obuf, b, _RS)
        stage_out(base + b, b, (sem0, sem1)[b])

    @pl.loop(2, _PER_W - 1, step=2)
    def _pair(k):
        for b in range(2):
            sem = (sem0, sem1)[b]
            wait_out(b, sem)
            stage_in(base + k + b)
            _tr_blocks(ibuf, obuf, b, _RS)
            stage_out(base + k + b, b, sem)

    wait_out(0, sem0)
    stage_in(base + _PER_W - 1)
    _tr_blocks(ibuf, obuf, 0, _RS)
    stage_out(base + _PER_W - 1, 0, sem0)

    @pl.when(wid == 0)
    def _extra():
        wait_out(1, sem1)
        stage_in(_NW * _PER_W)
        _tr_blocks(ibuf, obuf, 1, _RS)
        stage_out(_NW * _PER_W, 1, sem1)

    @pl.when(wid == 1)
    def _tail():
        wait_out(1, sem1)
        pltpu.sync_copy(wtail_hbm, tbuf)
        _tr_blocks(tbuf, obuf, 1, _TAIL_N)
        nw = _TAIL_N * _D
        pltpu.async_copy(
            obuf.at[1, pl.ds(0, nw)],
            w3_hbm.at[pl.ds(_TAIL_R0 * _D, nw)],
            sem1,
        ).wait()

    wait_out(0, sem0)
    pl.when(wid != 1)(lambda: wait_out(1, sem1))


def _emb_body(idx_hbm, table_hbm, out_hbm, idx_v, rows_v, *gsems):
    wid = lax.axis_index("s") * _NC + lax.axis_index("c")
    chunks = idx_hbm.shape[1]
    pltpu.sync_copy(idx_hbm.at[wid], idx_v)
    row0 = wid * chunks * _K

    @pl.loop(0, chunks, step=_NBUF)
    def _group(g0):
        cps = []
        for b in range(_NBUF):
            g = g0 + b
            cps.append(
                pltpu.async_copy(table_hbm.at[idx_v.at[g]], rows_v.at[b], gsems[b])
            )
        for b in range(_NBUF):
            cps[b].wait()
            pltpu.sync_copy(
                rows_v.at[b], out_hbm.at[pl.ds(row0 + (g0 + b) * _K, _K)]
            )


def kernel(x, weight):
    b, s = x.shape
    v, d = weight.shape
    assert (v, d) == (_V, _D)
    n = b * s
    assert n % (_NW * _K) == 0
    chunks = n // (_NW * _K)

    mesh = plsc.VectorSubcoreMesh(core_axis_name="c", subcore_axis_name="s")

    detr = pl.kernel(
        _detr_body,
        out_type=jax.ShapeDtypeStruct((_V * _D,), jnp.float32),
        mesh=mesh,
        scratch_types=[
            pltpu.VMEM((_D, _RS), jnp.float32),
            pltpu.VMEM((_D, _TAIL_N), jnp.float32),
            pltpu.VMEM((2, _SLABW), jnp.float32),
            pltpu.SemaphoreType.DMA,
            pltpu.SemaphoreType.DMA,
        ],
        compiler_params=pltpu.CompilerParams(
            use_tc_tiling_on_sc=True, needs_layout_passes=False
        ),
    )
    wtail_t = lax.slice(weight, (_TAIL_R0, 0), (_V, _D)).T
    w3 = detr(weight.T, wtail_t)
    wrm = w3.reshape(_V, _D)

    flat = x.reshape(-1).astype(jnp.int32)
    idx3 = flat.reshape(_NW, chunks, _K)

    run = pl.kernel(
        _emb_body,
        out_type=jax.ShapeDtypeStruct((n, d), jnp.float32),
        mesh=mesh,
        scratch_types=[
            pltpu.VMEM((chunks, _K), jnp.int32),
            pltpu.VMEM((_NBUF, _K, d), jnp.float32),
        ]
        + [pltpu.SemaphoreType.DMA] * _NBUF,
        compiler_params=pltpu.CompilerParams(use_tc_tiling_on_sc=False),
    )
    out = run(idx3, wrm)
    return out.reshape(b, s, d)


# speedup vs baseline: 3.1664x; 1.2330x over previous
import functools

import jax
import jax.numpy as jnp
from jax import lax
from jax.experimental import pallas as pl
from jax.experimental.pallas import tpu as pltpu
from jax.experimental.pallas import tpu_sc as plsc

_NC = 2
_NS = 16
_NW = _NC * _NS
_K = 128
_NBUF = 4

_V = 1000000
_D = 64
_RS = 384
_FULL = 2604
_TAIL_R0 = _FULL * _RS
_TAIL_N = _V - _TAIL_R0
_SLABW = _RS * _D


def _tr_blocks(src, lead, obuf, par, nrows):
    iot = lax.iota(jnp.int32, 16)
    rots = [(iot + d) % 16 for d in range(16)]
    rv64 = [rots[d] * 64 + iot for d in range(16)]
    bufv = jnp.full((16,), par, jnp.int32)
    leadv = [jnp.full((16,), lead, jnp.int32)] if lead is not None else []

    @pl.loop(0, nrows, step=16)
    def _blk(q0):
        qv = jnp.full((16,), q0, jnp.int32)

        @pl.loop(0, _D, step=16)
        def _cblk(c0):
            civ = iot + c0
            base = q0 * 64 + c0
            vs = {}
            for d in range(16):
                vs[d] = plsc.load_gather(src, leadv + [civ, qv + rots[d]])
                if d >= 2:
                    plsc.store_scatter(obuf, [bufv, rv64[d - 2] + base], vs.pop(d - 2))
            for d in (14, 15):
                plsc.store_scatter(obuf, [bufv, rv64[d] + base], vs.pop(d))


def _detr_body(wt_hbm, wtail_hbm, w3_hbm, ibuf, tbuf, obuf,
               isem0, isem1, osem0, osem1):
    wid = lax.axis_index("s") * _NC + lax.axis_index("c")
    start = wid * 81 + jnp.minimum(wid, 12)
    n = 81 + (wid < 12).astype(jnp.int32)

    def in_start(slab, par):
        for b, sem in ((0, isem0), (1, isem1)):
            @pl.when(par == b)
            def _(b=b, sem=sem):
                pltpu.async_copy(
                    wt_hbm.at[:, pl.ds(slab * _RS, _RS)], ibuf.at[b], sem)

    def in_wait(par):
        for b, sem in ((0, isem0), (1, isem1)):
            @pl.when(par == b)
            def _(b=b, sem=sem):
                pltpu.make_async_copy(
                    wt_hbm.at[:, pl.ds(0, _RS)], ibuf.at[b], sem).wait()

    def out_start(slab, par):
        for b, sem in ((0, osem0), (1, osem1)):
            @pl.when(par == b)
            def _(b=b, sem=sem):
                pltpu.async_copy(
                    obuf.at[b], w3_hbm.at[pl.ds(slab * _SLABW, _SLABW)], sem)

    def out_wait(par):
        for b, sem in ((0, osem0), (1, osem1)):
            @pl.when(par == b)
            def _(b=b, sem=sem):
                pltpu.make_async_copy(
                    obuf.at[b], w3_hbm.at[pl.ds(0, _SLABW)], sem).wait()

    in_start(start, 0)

    @pl.loop(0, 82)
    def _slab(k):
        @pl.when(k < n)
        def _do():
            par = k % 2
            pl.when(k + 1 < n)(lambda: in_start(start + k + 1, (k + 1) % 2))
            in_wait(par)
            pl.when(k >= 2)(lambda: out_wait(par))
            _tr_blocks(ibuf, par, obuf, par, _RS)
            out_start(start + k, par)

    out_wait((n - 2) % 2)
    out_wait((n - 1) % 2)

    @pl.when(wid == 1)
    def _tail():
        pltpu.sync_copy(wtail_hbm, tbuf)
        _tr_blocks(tbuf, None, obuf, 0, _TAIL_N)
        nw = _TAIL_N * _D
        pltpu.async_copy(
            obuf.at[0, pl.ds(0, nw)],
            w3_hbm.at[pl.ds(_TAIL_R0 * _D, nw)],
            osem0,
        ).wait()


def _emb_body(idx_hbm, table_hbm, out_hbm, idx_v, rows_v, *gsems):
    wid = lax.axis_index("s") * _NC + lax.axis_index("c")
    chunks = idx_hbm.shape[1]
    pltpu.sync_copy(idx_hbm.at[wid], idx_v)
    row0 = wid * chunks * _K

    @pl.loop(0, chunks, step=_NBUF)
    def _group(g0):
        cps = []
        for b in range(_NBUF):
            g = g0 + b
            cps.append(
                pltpu.async_copy(table_hbm.at[idx_v.at[g]], rows_v.at[b], gsems[b])
            )
        for b in range(_NBUF):
            cps[b].wait()
            pltpu.sync_copy(
                rows_v.at[b], out_hbm.at[pl.ds(row0 + (g0 + b) * _K, _K)]
            )


def kernel(x, weight):
    b, s = x.shape
    v, d = weight.shape
    assert (v, d) == (_V, _D)
    n = b * s
    assert n % (_NW * _K) == 0
    chunks = n // (_NW * _K)

    mesh = plsc.VectorSubcoreMesh(core_axis_name="c", subcore_axis_name="s")

    detr = pl.kernel(
        _detr_body,
        out_type=jax.ShapeDtypeStruct((_V * _D,), jnp.float32),
        mesh=mesh,
        scratch_types=[
            pltpu.VMEM((2, _D, _RS), jnp.float32),
            pltpu.VMEM((_D, _TAIL_N), jnp.float32),
            pltpu.VMEM((2, _SLABW), jnp.float32),
            pltpu.SemaphoreType.DMA,
            pltpu.SemaphoreType.DMA,
            pltpu.SemaphoreType.DMA,
            pltpu.SemaphoreType.DMA,
        ],
        compiler_params=pltpu.CompilerParams(
            use_tc_tiling_on_sc=True, needs_layout_passes=False
        ),
    )
    wtail_t = lax.slice(weight, (_TAIL_R0, 0), (_V, _D)).T
    w3 = detr(weight.T, wtail_t)
    wrm = w3.reshape(_V, _D)

    flat = x.reshape(-1).astype(jnp.int32)
    idx3 = flat.reshape(_NW, chunks, _K)

    run = pl.kernel(
        _emb_body,
        out_type=jax.ShapeDtypeStruct((n, d), jnp.float32),
        mesh=mesh,
        scratch_types=[
            pltpu.VMEM((chunks, _K), jnp.int32),
            pltpu.VMEM((_NBUF, _K, d), jnp.float32),
        ]
        + [pltpu.SemaphoreType.DMA] * _NBUF,
        compiler_params=pltpu.CompilerParams(use_tc_tiling_on_sc=False),
    )
    out = run(idx3, wrm)
    return out.reshape(b, s, d)


# speedup vs baseline: 3.2659x; 1.0314x over previous
import functools

import jax
import jax.numpy as jnp
from jax import lax
from jax.experimental import pallas as pl
from jax.experimental.pallas import tpu as pltpu
from jax.experimental.pallas import tpu_sc as plsc

_NC = 2
_NS = 16
_NW = _NC * _NS
_K = 128
_NBUF = 8

_V = 1000000
_D = 64
_RS = 384
_FULL = 2604
_TAIL_R0 = _FULL * _RS
_TAIL_N = _V - _TAIL_R0
_SLABW = _RS * _D


def _tr_blocks(src, lead, obuf, par, nrows):
    iot = lax.iota(jnp.int32, 16)
    rots = [(iot + d) % 16 for d in range(16)]
    rv64 = [rots[d] * 64 + iot for d in range(16)]
    bufv = jnp.full((16,), par, jnp.int32)
    leadv = [jnp.full((16,), lead, jnp.int32)] if lead is not None else []

    @pl.loop(0, nrows, step=16)
    def _blk(q0):
        qv = jnp.full((16,), q0, jnp.int32)

        @pl.loop(0, _D, step=16, unroll=2)
        def _cblk(c0):
            civ = iot + c0
            base = q0 * 64 + c0
            vs = {}
            for d in range(16):
                vs[d] = plsc.load_gather(src, leadv + [civ, qv + rots[d]])
                if d >= 2:
                    plsc.store_scatter(obuf, [bufv, rv64[d - 2] + base], vs.pop(d - 2))
            for d in (14, 15):
                plsc.store_scatter(obuf, [bufv, rv64[d] + base], vs.pop(d))


def _detr_body(wt_hbm, wtail_hbm, w3_hbm, ibuf, tbuf, obuf,
               isem0, isem1, osem0, osem1):
    wid = lax.axis_index("s") * _NC + lax.axis_index("c")
    start = wid * 81 + jnp.minimum(wid, 12)
    n = 81 + (wid < 12).astype(jnp.int32)

    def in_start(slab, par):
        for b, sem in ((0, isem0), (1, isem1)):
            @pl.when(par == b)
            def _(b=b, sem=sem):
                pltpu.async_copy(
                    wt_hbm.at[:, pl.ds(slab * _RS, _RS)], ibuf.at[b], sem)

    def in_wait(par):
        for b, sem in ((0, isem0), (1, isem1)):
            @pl.when(par == b)
            def _(b=b, sem=sem):
                pltpu.make_async_copy(
                    wt_hbm.at[:, pl.ds(0, _RS)], ibuf.at[b], sem).wait()

    def out_start(slab, par):
        for b, sem in ((0, osem0), (1, osem1)):
            @pl.when(par == b)
            def _(b=b, sem=sem):
                pltpu.async_copy(
                    obuf.at[b], w3_hbm.at[pl.ds(slab * _SLABW, _SLABW)], sem)

    def out_wait(par):
        for b, sem in ((0, osem0), (1, osem1)):
            @pl.when(par == b)
            def _(b=b, sem=sem):
                pltpu.make_async_copy(
                    obuf.at[b], w3_hbm.at[pl.ds(0, _SLABW)], sem).wait()

    in_start(start, 0)

    @pl.loop(0, 82)
    def _slab(k):
        @pl.when(k < n)
        def _do():
            par = k % 2
            pl.when(k + 1 < n)(lambda: in_start(start + k + 1, (k + 1) % 2))
            in_wait(par)
            pl.when(k >= 2)(lambda: out_wait(par))
            _tr_blocks(ibuf, par, obuf, par, _RS)
            out_start(start + k, par)

    out_wait((n - 2) % 2)
    out_wait((n - 1) % 2)

    @pl.when(wid == 1)
    def _tail():
        pltpu.sync_copy(wtail_hbm, tbuf)
        _tr_blocks(tbuf, None, obuf, 0, _TAIL_N)
        nw = _TAIL_N * _D
        pltpu.async_copy(
            obuf.at[0, pl.ds(0, nw)],
            w3_hbm.at[pl.ds(_TAIL_R0 * _D, nw)],
            osem0,
        ).wait()


def _emb_body(idx_hbm, table_hbm, out_hbm, idx_v, rows_v, *gsems):
    wid = lax.axis_index("s") * _NC + lax.axis_index("c")
    chunks = idx_hbm.shape[1]
    pltpu.sync_copy(idx_hbm.at[wid], idx_v)
    row0 = wid * chunks * _K

    @pl.loop(0, chunks, step=_NBUF)
    def _group(g0):
        cps = []
        for b in range(_NBUF):
            g = g0 + b
            cps.append(
                pltpu.async_copy(table_hbm.at[idx_v.at[g]], rows_v.at[b], gsems[b])
            )
        for b in range(_NBUF):
            cps[b].wait()
            pltpu.sync_copy(
                rows_v.at[b], out_hbm.at[pl.ds(row0 + (g0 + b) * _K, _K)]
            )


def kernel(x, weight):
    b, s = x.shape
    v, d = weight.shape
    assert (v, d) == (_V, _D)
    n = b * s
    assert n % (_NW * _K) == 0
    chunks = n // (_NW * _K)

    mesh = plsc.VectorSubcoreMesh(core_axis_name="c", subcore_axis_name="s")

    detr = pl.kernel(
        _detr_body,
        out_type=jax.ShapeDtypeStruct((_V * _D,), jnp.float32),
        mesh=mesh,
        scratch_types=[
            pltpu.VMEM((2, _D, _RS), jnp.float32),
            pltpu.VMEM((_D, _TAIL_N), jnp.float32),
            pltpu.VMEM((2, _SLABW), jnp.float32),
            pltpu.SemaphoreType.DMA,
            pltpu.SemaphoreType.DMA,
            pltpu.SemaphoreType.DMA,
            pltpu.SemaphoreType.DMA,
        ],
        compiler_params=pltpu.CompilerParams(
            use_tc_tiling_on_sc=True, needs_layout_passes=False
        ),
    )
    wtail_t = lax.slice(weight, (_TAIL_R0, 0), (_V, _D)).T
    w3 = detr(weight.T, wtail_t)
    wrm = w3.reshape(_V, _D)

    flat = x.reshape(-1).astype(jnp.int32)
    idx3 = flat.reshape(_NW, chunks, _K)

    run = pl.kernel(
        _emb_body,
        out_type=jax.ShapeDtypeStruct((n, d), jnp.float32),
        mesh=mesh,
        scratch_types=[
            pltpu.VMEM((chunks, _K), jnp.int32),
            pltpu.VMEM((_NBUF, _K, d), jnp.float32),
        ]
        + [pltpu.SemaphoreType.DMA] * _NBUF,
        compiler_params=pltpu.CompilerParams(use_tc_tiling_on_sc=False),
    )
    out = run(idx3, wrm)
    return out.reshape(b, s, d)


# speedup vs baseline: 3.2869x; 1.0064x over previous
import functools

import jax
import jax.numpy as jnp
from jax import lax
from jax.experimental import pallas as pl
from jax.experimental.pallas import tpu as pltpu
from jax.experimental.pallas import tpu_sc as plsc

_NC = 2
_NS = 16
_NW = _NC * _NS
_K = 128
_NBUF = 8

_V = 1000000
_D = 64
_RS = 384
_FULL = 2604
_TAIL_R0 = _FULL * _RS
_TAIL_N = _V - _TAIL_R0
_SLABW = _RS * _D


def _tr_blocks(src, lead, obuf, par, nrows):
    iot = lax.iota(jnp.int32, 16)
    rots = [(iot + d) % 16 for d in range(16)]
    rv64 = [rots[d] * 64 + iot for d in range(16)]
    bufv = jnp.full((16,), par, jnp.int32)
    leadv = [jnp.full((16,), lead, jnp.int32)] if lead is not None else []

    @pl.loop(0, nrows, step=16, unroll=2)
    def _blk(q0):
        qv = jnp.full((16,), q0, jnp.int32)

        @pl.loop(0, _D, step=16, unroll=2)
        def _cblk(c0):
            civ = iot + c0
            base = q0 * 64 + c0
            vs = {}
            for d in range(16):
                vs[d] = plsc.load_gather(src, leadv + [civ, qv + rots[d]])
                if d >= 2:
                    plsc.store_scatter(obuf, [bufv, rv64[d - 2] + base], vs.pop(d - 2))
            for d in (14, 15):
                plsc.store_scatter(obuf, [bufv, rv64[d] + base], vs.pop(d))


def _detr_body(wt_hbm, wtail_hbm, w3_hbm, ibuf, tbuf, obuf,
               isem0, isem1, osem0, osem1):
    wid = lax.axis_index("s") * _NC + lax.axis_index("c")
    start = wid * 81 + jnp.minimum(wid, 12)
    n = 81 + (wid < 12).astype(jnp.int32)

    def in_start(slab, par):
        for b, sem in ((0, isem0), (1, isem1)):
            @pl.when(par == b)
            def _(b=b, sem=sem):
                pltpu.async_copy(
                    wt_hbm.at[:, pl.ds(slab * _RS, _RS)], ibuf.at[b], sem)

    def in_wait(par):
        for b, sem in ((0, isem0), (1, isem1)):
            @pl.when(par == b)
            def _(b=b, sem=sem):
                pltpu.make_async_copy(
                    wt_hbm.at[:, pl.ds(0, _RS)], ibuf.at[b], sem).wait()

    def out_start(slab, par):
        for b, sem in ((0, osem0), (1, osem1)):
            @pl.when(par == b)
            def _(b=b, sem=sem):
                pltpu.async_copy(
                    obuf.at[b], w3_hbm.at[pl.ds(slab * _SLABW, _SLABW)], sem)

    def out_wait(par):
        for b, sem in ((0, osem0), (1, osem1)):
            @pl.when(par == b)
            def _(b=b, sem=sem):
                pltpu.make_async_copy(
                    obuf.at[b], w3_hbm.at[pl.ds(0, _SLABW)], sem).wait()

    in_start(start, 0)

    @pl.loop(0, 82)
    def _slab(k):
        @pl.when(k < n)
        def _do():
            par = k % 2
            pl.when(k + 1 < n)(lambda: in_start(start + k + 1, (k + 1) % 2))
            in_wait(par)
            pl.when(k >= 2)(lambda: out_wait(par))
            _tr_blocks(ibuf, par, obuf, par, _RS)
            out_start(start + k, par)

    out_wait((n - 2) % 2)
    out_wait((n - 1) % 2)

    @pl.when(wid == 1)
    def _tail():
        pltpu.sync_copy(wtail_hbm, tbuf)
        _tr_blocks(tbuf, None, obuf, 0, _TAIL_N)
        nw = _TAIL_N * _D
        pltpu.async_copy(
            obuf.at[0, pl.ds(0, nw)],
            w3_hbm.at[pl.ds(_TAIL_R0 * _D, nw)],
            osem0,
        ).wait()


def _emb_body(idx_hbm, table_hbm, out_hbm, idx_v, rows_v, *gsems):
    wid = lax.axis_index("s") * _NC + lax.axis_index("c")
    chunks = idx_hbm.shape[1]
    pltpu.sync_copy(idx_hbm.at[wid], idx_v)
    row0 = wid * chunks * _K

    @pl.loop(0, chunks, step=_NBUF)
    def _group(g0):
        cps = []
        for b in range(_NBUF):
            g = g0 + b
            cps.append(
                pltpu.async_copy(table_hbm.at[idx_v.at[g]], rows_v.at[b], gsems[b])
            )
        for b in range(_NBUF):
            cps[b].wait()
            pltpu.sync_copy(
                rows_v.at[b], out_hbm.at[pl.ds(row0 + (g0 + b) * _K, _K)]
            )


def kernel(x, weight):
    b, s = x.shape
    v, d = weight.shape
    assert (v, d) == (_V, _D)
    n = b * s
    assert n % (_NW * _K) == 0
    chunks = n // (_NW * _K)

    mesh = plsc.VectorSubcoreMesh(core_axis_name="c", subcore_axis_name="s")

    detr = pl.kernel(
        _detr_body,
        out_type=jax.ShapeDtypeStruct((_V * _D,), jnp.float32),
        mesh=mesh,
        scratch_types=[
            pltpu.VMEM((2, _D, _RS), jnp.float32),
            pltpu.VMEM((_D, _TAIL_N), jnp.float32),
            pltpu.VMEM((2, _SLABW), jnp.float32),
            pltpu.SemaphoreType.DMA,
            pltpu.SemaphoreType.DMA,
            pltpu.SemaphoreType.DMA,
            pltpu.SemaphoreType.DMA,
        ],
        compiler_params=pltpu.CompilerParams(
            use_tc_tiling_on_sc=True, needs_layout_passes=False
        ),
    )
    wtail_t = lax.slice(weight, (_TAIL_R0, 0), (_V, _D)).T
    w3 = detr(weight.T, wtail_t)
    wrm = w3.reshape(_V, _D)

    flat = x.reshape(-1).astype(jnp.int32)
    idx3 = flat.reshape(_NW, chunks, _K)

    run = pl.kernel(
        _emb_body,
        out_type=jax.ShapeDtypeStruct((n, d), jnp.float32),
        mesh=mesh,
        scratch_types=[
            pltpu.VMEM((chunks, _K), jnp.int32),
            pltpu.VMEM((_NBUF, _K, d), jnp.float32),
        ]
        + [pltpu.SemaphoreType.DMA] * _NBUF,
        compiler_params=pltpu.CompilerParams(use_tc_tiling_on_sc=False),
    )
    out = run(idx3, wrm)
    return out.reshape(b, s, d)
